# baseline (speedup 1.0000x reference)
import functools

import jax
import jax.numpy as jnp
from jax import lax
from jax.experimental import pallas as pl
from jax.experimental.pallas import tpu as pltpu
from jax.experimental.pallas import tpu_sc as plsc

D = 32
DP = 128
S = 26
B = 16384
CHUNK = 128
NCHUNKS_TOT = (S * B) // CHUNK
NC = 2
NS = 16
NW = NC * NS
CPW = NCHUNKS_TOT // NW
NBUF = 2

_mesh = plsc.VectorSubcoreMesh(core_axis_name="c", subcore_axis_name="s")


@functools.partial(
    pl.kernel,
    mesh=_mesh,
    out_type=jax.ShapeDtypeStruct((S, D, B), jnp.float32),
    scratch_types=[
        pltpu.VMEM((CPW, CHUNK), jnp.int32),
        pltpu.VMEM((NBUF, CHUNK, DP), jnp.float32),
        pltpu.VMEM((NBUF, D, CHUNK), jnp.float32),
        [pltpu.SemaphoreType.DMA] * NBUF,
        [pltpu.SemaphoreType.DMA] * NBUF,
    ],
    compiler_params=pltpu.CompilerParams(use_tc_tiling_on_sc=True,
                                         needs_layout_passes=False),
)
def _gather_kernel(idx_hbm, table_hbm, out_hbm, idx_v, bufg, buft, sg, sw):
    wid = lax.axis_index("s") * NC + lax.axis_index("c")
    c0 = wid * CPW
    pltpu.sync_copy(idx_hbm.at[pl.ds(c0, CPW)], idx_v)

    def gather(j, b):
        return pltpu.make_async_copy(
            table_hbm.at[idx_v.at[j]], bufg.at[b], sg[b])

    def write(j, b):
        cg = c0 + j
        s = cg // CHUNK
        bb = (cg % CHUNK) * CHUNK
        return pltpu.make_async_copy(
            buft.at[b], out_hbm.at[s, :, pl.ds(bb, CHUNK)], sw[b])

    def transpose(b):
        src = bufg.at[b]
        for v in range(CHUNK // 16):
            rows = jnp.arange(16, dtype=jnp.int32) + 16 * v
            for c in range(D):
                cols = jnp.full((16,), c, dtype=jnp.int32)
                vreg = plsc.load_gather(src, [rows, cols])
                buft[b, c, pl.ds(16 * v, 16)] = vreg

    gather(0, 0).start()
    gather(1, 1).start()
    for b in range(NBUF):
        gather(b, b).wait()
        transpose(b)
        write(b, b).start()
        gather(b + NBUF, b).start()

    def group(g, carry):
        for b in range(NBUF):
            j = g * NBUF + b
            gather(j, b).wait()
            write(j - NBUF, b).wait()
            transpose(b)
            write(j, b).start()
            gather(j + NBUF, b).start()
        return carry

    lax.fori_loop(1, CPW // NBUF - 1, group, 0)

    for b in range(NBUF):
        j = CPW - NBUF + b
        gather(j, b).wait()
        write(j - NBUF, b).wait()
        transpose(b)
        write(j, b).start()
    for b in range(NBUF):
        write(CPW - NBUF + b, b).wait()


def kernel(token_ids, weight):
    wpad = jnp.pad(weight.astype(jnp.float32), ((0, 0), (0, DP - D)))
    ids = token_ids.astype(jnp.int32).T.reshape(NCHUNKS_TOT, CHUNK)
    out_t = _gather_kernel(ids, wpad)
    return out_t.transpose(2, 0, 1)
